# baseline (speedup 1.0000x reference)
import functools

import jax
import jax.numpy as jnp
from jax import lax
from jax.experimental import pallas as pl
from jax.experimental.pallas import tpu as pltpu
from jax.experimental.pallas import tpu_sc as plsc

N = 10000
E = 320000
L = 128
EMB = 128
NREL = 4
TXT = 256

NC = 2
NS = 16
NW = NC * NS

NPW = (N + NW - 1) // NW
EB = 128
EROWS = 2560
EPAD = EROWS * EB
RPT = EROWS // NS
SLABS = RPT // 16

DHALF = 5000
EACC = 5248
EDUMMY = 5120
ETR = EACC // NS

CR = 1024
CACC = 4224
CDUMMY = 4096
CTR = CACC // NS
NPD = 10 * CR

_MESH = plsc.VectorSubcoreMesh(core_axis_name="c", subcore_axis_name="s")


def _nrsqrt(t):
    y = lax.bitcast_convert_type(t, jnp.int32)
    y = jnp.int32(0x5F3759DF) - (y >> 1)
    y = lax.bitcast_convert_type(y, jnp.float32)
    for _ in range(3):
        y = y * (1.5 - 0.5 * t * y * y)
    return y


@functools.partial(
    pl.kernel,
    mesh=_MESH,
    out_type=jax.ShapeDtypeStruct((N, L, 16), jnp.float32),
    scratch_types=[
        pltpu.VMEM((L,), jnp.int32),
        pltpu.VMEM((L,), jnp.int32),
        pltpu.VMEM((L, EMB), jnp.float32),
        pltpu.VMEM((L, EMB), jnp.float32),
        pltpu.VMEM((L, 16), jnp.float32),
        pltpu.VMEM((L, 16), jnp.float32),
        pltpu.VMEM((EMB,), jnp.float32),
        pltpu.SemaphoreType.DMA,
        pltpu.SemaphoreType.DMA,
        pltpu.SemaphoreType.DMA,
        pltpu.SemaphoreType.DMA,
    ],
)
def _sc_embed(tab_hbm, src_hbm, w_hbm, out_hbm,
              idx0, idx1, S0, S1, P0, P1, w_v, semS0, semS1, semP0, semP1):
    core = lax.axis_index("c")
    sub = lax.axis_index("s")
    wid = core * NS + sub
    base = wid * NPW
    num = jnp.maximum(0, jnp.minimum(NPW, N - base))
    idx = [idx0, idx1]
    S = [S0, S1]
    P = [P0, P1]
    semS = [semS0, semS1]
    semP = [semP0, semP1]

    pltpu.sync_copy(w_hbm, w_v)

    def load_idx_and_gather(i, p):
        pltpu.sync_copy(src_hbm.at[pl.ds((base + i) * L, L)], idx[p])
        pltpu.async_copy(tab_hbm.at[idx[p]], S[p], semS[p])

    @pl.when(num > 0)
    def _():
        load_idx_and_gather(0, 0)

    @pl.when(num > 1)
    def _():
        load_idx_and_gather(1, 1)

    def pair_body(k, carry):
        for p in range(2):
            i = k * 2 + p

            @pl.when(i < num)
            def _():
                node = base + i
                Sp, Pp = S[p], P[p]
                pltpu.make_async_copy(tab_hbm.at[idx[p]], Sp, semS[p]).wait()

                @pl.when(i >= 2)
                def _():
                    pltpu.make_async_copy(Pp, out_hbm.at[node], semP[p]).wait()

                def ss_row(r2, accs):
                    out = accs
                    for rr in range(2):
                        r = r2 * 2 + rr
                        out = tuple(
                            out[j] + Sp[r, pl.ds(j * 16, 16)] * Sp[r, pl.ds(j * 16, 16)]
                            for j in range(8)
                        )
                    return out
                accs = lax.fori_loop(0, L // 2, ss_row,
                                     tuple(jnp.zeros((16,), jnp.float32) for _ in range(8)))

                cs = []
                for j in range(8):
                    wv = w_v[pl.ds(j * 16, 16)]
                    t = wv * wv * accs[j]
                    norm = t * _nrsqrt(t)
                    cs.append(wv / jnp.maximum(norm, 1e-12))

                def p_row(r2, carry2):
                    for rr in range(2):
                        r = r2 * 2 + rr
                        pp = Sp[r, pl.ds(0, 16)] * cs[0]
                        for j in range(1, 8):
                            pp = pp + Sp[r, pl.ds(j * 16, 16)] * cs[j]
                        Pp[r] = pp
                    return carry2
                lax.fori_loop(0, L // 2, p_row, 0)

                pltpu.async_copy(Pp, out_hbm.at[node], semP[p])

                @pl.when(i + 2 < num)
                def _():
                    load_idx_and_gather(i + 2, p)
        return carry

    lax.fori_loop(0, (NPW + 1) // 2, pair_body, 0)

    @pl.when(num >= 1)
    def _():
        pltpu.make_async_copy(P[0], out_hbm.at[base], semP[0]).wait()

    @pl.when(num >= 2)
    def _():
        pltpu.make_async_copy(P[1], out_hbm.at[base], semP[1]).wait()


@functools.partial(
    pl.kernel,
    mesh=_MESH,
    out_type=jax.ShapeDtypeStruct((NC, DHALF, EMB), jnp.float32),
    scratch_types=[
        pltpu.VMEM((16, EB), jnp.int32),
        pltpu.VMEM((16, EB), jnp.int32),
        pltpu.VMEM((16, EB), jnp.int32),
        pltpu.VMEM((16, EB), jnp.int32),
        pltpu.VMEM((16, EB), jnp.int32),
        pltpu.VMEM((16, EB), jnp.int32),
        pltpu.VMEM((EB, EMB), jnp.float32),
        pltpu.VMEM((EB, EMB), jnp.float32),
        pltpu.VMEM((EB, EMB), jnp.float32),
        pltpu.VMEM((EB, EMB), jnp.float32),
        pltpu.VMEM_SHARED((EACC, EMB), jnp.float32),
        pltpu.SemaphoreType.DMA,
        pltpu.SemaphoreType.DMA,
        pltpu.SemaphoreType.DMA,
        pltpu.SemaphoreType.DMA,
    ],
)
def _sc_edge(y_hbm, ic_hbm, vg_hbm, sg_hbm, se_hbm, zz_hbm, agg_hbm,
             vg0, vg1, sg0, sg1, se0, se1, rows0, rows1, sc0, sc1,
             acc_sh, semv0, semv1, sems0, sems1):
    core = lax.axis_index("c")
    sub = lax.axis_index("s")
    vg = [vg0, vg1]
    sg = [sg0, sg1]
    se = [se0, se1]
    rows = [rows0, rows1]
    scl = [sc0, sc1]
    semv = [semv0, semv1]
    sems = [sems0, sems1]

    pltpu.sync_copy(zz_hbm.at[pl.ds(sub * ETR, ETR)],
                    acc_sh.at[pl.ds(sub * ETR, ETR)])
    plsc.subcore_barrier()

    def load_slab(s, par):
        pltpu.sync_copy(vg_hbm.at[pl.ds(sub * RPT + s * 16, 16)], vg[par])
        pltpu.sync_copy(sg_hbm.at[pl.ds(sub * RPT + s * 16, 16)], sg[par])
        pltpu.sync_copy(se_hbm.at[core, pl.ds(sub * RPT + s * 16, 16)], se[par])

    def issue(vgref, sgref, p):
        pltpu.async_copy(y_hbm.at[vgref], rows[p], semv[p])
        pltpu.async_copy(ic_hbm.at[sgref], scl[p], sems[p])

    load_slab(0, 0)
    issue(vg0.at[0], sg0.at[0], 0)
    issue(vg0.at[1], sg0.at[1], 1)

    def slab_pair(sp, carry):
        for sq in range(2):
            s = sp * 2 + sq
            vgc, vgn = vg[sq], vg[1 - sq]
            sgc, sgn = sg[sq], sg[1 - sq]
            sec = se[sq]

            @pl.when(s < SLABS - 1)
            def _():
                load_slab(s + 1, 1 - sq)

            def bpair(bp, c2):
                for bq in range(2):
                    b = bp * 2 + bq
                    gb = s * 16 + b
                    pltpu.make_async_copy(y_hbm.at[vgc.at[b]], rows[bq], semv[bq]).wait()
                    pltpu.make_async_copy(ic_hbm.at[sgc.at[b]], scl[bq], sems[bq]).wait()

                    def mrow(r2, c3):
                        for rr in range(2):
                            r = r2 * 2 + rr
                            for j in range(EMB // 16):
                                rows[bq][r, pl.ds(j * 16, 16)] = (
                                    rows[bq][r, pl.ds(j * 16, 16)]
                                    * scl[bq][r, pl.ds(j * 16, 16)])
                        return c3
                    lax.fori_loop(0, EB // 2, mrow, 0)

                    pltpu.sync_copy(rows[bq], acc_sh.at[sec.at[b]], add=True)

                    @pl.when(bp < 7)
                    def _():
                        issue(vgc.at[b + 2], sgc.at[b + 2], bq)

                    @pl.when(jnp.logical_and(bp == 7, gb + 2 < SLABS * 16))
                    def _():
                        issue(vgn.at[bq], sgn.at[bq], bq)
                return c2

            lax.fori_loop(0, 8, bpair, 0)
        return carry

    lax.fori_loop(0, SLABS // 2, slab_pair, 0)

    plsc.subcore_barrier()

    @pl.when(sub < NS - 1)
    def _():
        pltpu.sync_copy(acc_sh.at[pl.ds(sub * ETR, ETR)],
                        agg_hbm.at[core, pl.ds(sub * ETR, ETR)])

    @pl.when(sub == NS - 1)
    def _():
        pltpu.sync_copy(acc_sh.at[pl.ds((NS - 1) * ETR, DHALF - (NS - 1) * ETR)],
                        agg_hbm.at[core, pl.ds((NS - 1) * ETR, DHALF - (NS - 1) * ETR)])


_BN = 1000


def _tc_fold_body(p_ref, g_ref, out_ref):
    out_ref[...] = jnp.dot(p_ref[...], g_ref[...],
                           preferred_element_type=jnp.float32)


def _tc_fold(p_flat, g):
    return pl.pallas_call(
        _tc_fold_body,
        grid=(N // _BN,),
        in_specs=[
            pl.BlockSpec((_BN * 16, L), lambda i: (i, 0)),
            pl.BlockSpec((L, 8), lambda i: (0, 0)),
        ],
        out_specs=pl.BlockSpec((_BN * 16, 8), lambda i: (i, 0)),
        out_shape=jax.ShapeDtypeStruct((N * 16, 8), jnp.float32),
    )(p_flat, g)


def _tc_inv_body(cnt_ref, out_ref):
    cnt = cnt_ref[...]
    for r in range(NREL):
        s = cnt[:, r * 32:(r + 1) * 32]
        inv = 1.0 / jnp.maximum(s, 1.0)
        out_ref[r] = jnp.concatenate([inv, inv, inv, inv], axis=1)


def _tc_inv(cnt_oh):
    return pl.pallas_call(
        _tc_inv_body,
        grid=(N // _BN,),
        in_specs=[pl.BlockSpec((_BN, EMB), lambda i: (i, 0))],
        out_specs=pl.BlockSpec((NREL, _BN, EMB), lambda i: (0, i, 0)),
        out_shape=jax.ShapeDtypeStruct((NREL, N, EMB), jnp.float32),
    )(cnt_oh)


def _tc_dense1_body(fse_ref, seg_ref, st_ref, w1_ref, b1_ref, w2_ref, b2_ref,
                    cw_ref, x0_ref, y_ref):
    f_se = fse_ref[...]
    seg = seg_ref[...]
    st = st_ref[...]
    st2 = st * st
    c0 = jnp.sum((seg == 0).astype(jnp.float32), axis=1, keepdims=True)
    c1 = jnp.sum((seg == 1).astype(jnp.float32), axis=1, keepdims=True)
    c2 = jnp.sum((seg == 2).astype(jnp.float32), axis=1, keepdims=True)
    q = c0 * st2[0:1, :] + c1 * st2[1:2, :] + c2 * st2[2:3, :]
    inv = 1.0 / jnp.maximum(jnp.sqrt(q), 1e-12)
    d0 = jnp.sum(inv * st[0:1, :], axis=1, keepdims=True)
    d1 = jnp.sum(inv * st[1:2, :], axis=1, keepdims=True)
    d2 = jnp.sum(inv * st[2:3, :], axis=1, keepdims=True)
    f_ge = jnp.where(seg == 0, d0, jnp.where(seg == 1, d1, d2))
    f = f_se + f_ge
    h = jnp.maximum(jnp.dot(f, w1_ref[...], preferred_element_type=jnp.float32, precision=lax.Precision.HIGHEST)
                    + b1_ref[...], 0.0)
    x0 = jnp.dot(h, w2_ref[...], preferred_element_type=jnp.float32, precision=lax.Precision.HIGHEST) + b2_ref[...]
    x0_ref[...] = x0
    for r in range(NREL):
        y_ref[r] = jnp.dot(x0, cw_ref[r], preferred_element_type=jnp.float32, precision=lax.Precision.HIGHEST)


def _tc_dense1(f_se, seg, seg_tab_p, q_W1, q_b1, q_W2, q_b2, c1_w):
    return pl.pallas_call(
        _tc_dense1_body,
        grid=(N // _BN,),
        in_specs=[
            pl.BlockSpec((_BN, L), lambda i: (i, 0)),
            pl.BlockSpec((_BN, L), lambda i: (i, 0)),
            pl.BlockSpec((8, EMB), lambda i: (0, 0)),
            pl.BlockSpec((EMB, TXT), lambda i: (0, 0)),
            pl.BlockSpec((1, TXT), lambda i: (0, 0)),
            pl.BlockSpec((TXT, EMB), lambda i: (0, 0)),
            pl.BlockSpec((1, EMB), lambda i: (0, 0)),
            pl.BlockSpec((NREL, EMB, EMB), lambda i: (0, 0, 0)),
        ],
        out_specs=[
            pl.BlockSpec((_BN, EMB), lambda i: (i, 0)),
            pl.BlockSpec((NREL, _BN, EMB), lambda i: (0, i, 0)),
        ],
        out_shape=[
            jax.ShapeDtypeStruct((N, EMB), jnp.float32),
            jax.ShapeDtypeStruct((NREL, N, EMB), jnp.float32),
        ],
    )(f_se, seg, seg_tab_p, q_W1, q_b1, q_W2, q_b2, c1_w)


def _tc_dense2_body(x_ref, agg_ref, root_ref, b_ref, cw_ref, x1_ref, y_ref):
    x1 = jnp.maximum(
        jnp.dot(x_ref[...], root_ref[...], preferred_element_type=jnp.float32, precision=lax.Precision.HIGHEST)
        + b_ref[...] + agg_ref[...], 0.0)
    x1_ref[...] = x1
    for r in range(NREL):
        y_ref[r] = jnp.dot(x1, cw_ref[r], preferred_element_type=jnp.float32, precision=lax.Precision.HIGHEST)


def _tc_dense2(x, agg, root, b, c2_w):
    return pl.pallas_call(
        _tc_dense2_body,
        grid=(N // _BN,),
        in_specs=[
            pl.BlockSpec((_BN, EMB), lambda i: (i, 0)),
            pl.BlockSpec((_BN, EMB), lambda i: (i, 0)),
            pl.BlockSpec((EMB, EMB), lambda i: (0, 0)),
            pl.BlockSpec((1, EMB), lambda i: (0, 0)),
            pl.BlockSpec((NREL, EMB, EMB), lambda i: (0, 0, 0)),
        ],
        out_specs=[
            pl.BlockSpec((_BN, EMB), lambda i: (i, 0)),
            pl.BlockSpec((NREL, _BN, EMB), lambda i: (0, i, 0)),
        ],
        out_shape=[
            jax.ShapeDtypeStruct((N, EMB), jnp.float32),
            jax.ShapeDtypeStruct((NREL, N, EMB), jnp.float32),
        ],
    )(x, agg, root, b, c2_w)


def _tc_dense3_body(x_ref, agg_ref, root_ref, b_ref, lw_ref, lb_ref, out_ref):
    x2 = jnp.maximum(
        jnp.dot(x_ref[...], root_ref[...], preferred_element_type=jnp.float32, precision=lax.Precision.HIGHEST)
        + b_ref[...] + agg_ref[...], 0.0)
    out_ref[...] = (jnp.dot(x2, lw_ref[...], preferred_element_type=jnp.float32, precision=lax.Precision.HIGHEST)
                    + lb_ref[...])


def _tc_dense3(x, agg, root, b, lin_W, lin_b):
    return pl.pallas_call(
        _tc_dense3_body,
        grid=(N // _BN,),
        in_specs=[
            pl.BlockSpec((_BN, EMB), lambda i: (i, 0)),
            pl.BlockSpec((_BN, EMB), lambda i: (i, 0)),
            pl.BlockSpec((EMB, EMB), lambda i: (0, 0)),
            pl.BlockSpec((1, EMB), lambda i: (0, 0)),
            pl.BlockSpec((EMB, EMB), lambda i: (0, 0)),
            pl.BlockSpec((1, EMB), lambda i: (0, 0)),
        ],
        out_specs=pl.BlockSpec((_BN, EMB), lambda i: (i, 0)),
        out_shape=jax.ShapeDtypeStruct((N, EMB), jnp.float32),
    )(x, agg, root, b, lin_W, lin_b)


def kernel(src_tab, seg_tab, w, q_W1, q_b1, q_W2, q_b2,
           c1_w, c1_root, c1_b, c2_w, c2_root, c2_b, lin_W, lin_b,
           src, seg, edge_index, edge_type):
    src = src.astype(jnp.int32)
    seg = seg.astype(jnp.int32)
    esrc = edge_index[0].astype(jnp.int32)
    edst = edge_index[1].astype(jnp.int32)
    etype = edge_type.astype(jnp.int32)

    padi = jnp.zeros((EPAD - E,), jnp.int32)
    vgidx = jnp.concatenate([etype * N + esrc, padi]).reshape(EROWS, EB)
    sgidx = jnp.concatenate([etype * N + edst, padi]).reshape(EROWS, EB)
    spread = (jnp.arange(EPAD, dtype=jnp.int32) % 5000) * 8
    cgv = (jnp.concatenate([etype, jnp.full((EPAD - E,), 4, jnp.int32)])
           + spread).reshape(EROWS, EB)
    cgs = (spread + 4).reshape(EROWS, EB)
    se_l = [jnp.concatenate(
        [jnp.where((edst >= h * DHALF) & (edst < (h + 1) * DHALF),
                   edst - h * DHALF, EDUMMY),
         jnp.full((EPAD - E,), EDUMMY, jnp.int32)]) for h in range(NC)]
    seidx = jnp.stack(se_l).reshape(NC, EROWS, EB)

    oh8 = jnp.where(jnp.arange(8, dtype=jnp.int32)[:, None] < 4,
                    (jnp.arange(EMB, dtype=jnp.int32)[None, :] // 32
                     == jnp.arange(8, dtype=jnp.int32)[:, None]).astype(jnp.float32),
                    1.0)
    oh_full = jnp.tile(oh8, (5000, 1))
    zz = jnp.zeros((EACC, EMB), jnp.float32)

    seg_tab_p = jnp.pad(seg_tab, ((0, 5), (0, 0)))
    b1 = q_b1.reshape(1, TXT)
    b2 = q_b2.reshape(1, EMB)
    c1_b_r = c1_b.reshape(1, EMB)
    c2_b_r = c2_b.reshape(1, EMB)
    lin_b_r = lin_b.reshape(1, EMB)

    p_out = _sc_embed(src_tab, src.reshape(N * L), w)
    g = (jnp.arange(L, dtype=jnp.int32)[:, None] // 16
         == jnp.arange(8, dtype=jnp.int32)[None, :]).astype(jnp.float32)
    f_se = _tc_fold(p_out.reshape(N * 16, L), g).reshape(N, L)

    cnt_oh = _sc_edge(oh_full, oh_full, cgv, cgs, seidx, zz)
    ic = _tc_inv(cnt_oh.reshape(N, EMB)).reshape(NREL * N, EMB)

    x0, y0 = _tc_dense1(f_se, seg, seg_tab_p, q_W1, b1, q_W2, b2, c1_w)
    agg1 = _sc_edge(y0.reshape(NREL * N, EMB), ic, vgidx, sgidx, seidx, zz)
    x1, y1 = _tc_dense2(x0, agg1.reshape(N, EMB), c1_root, c1_b_r, c2_w)
    agg2 = _sc_edge(y1.reshape(NREL * N, EMB), ic, vgidx, sgidx, seidx, zz)
    return _tc_dense3(x1, agg2.reshape(N, EMB), c2_root, c2_b_r, lin_W, lin_b_r)


# speedup vs baseline: 1.0320x; 1.0320x over previous
import functools

import jax
import jax.numpy as jnp
from jax import lax
from jax.experimental import pallas as pl
from jax.experimental.pallas import tpu as pltpu
from jax.experimental.pallas import tpu_sc as plsc

N = 10000
E = 320000
L = 128
EMB = 128
NREL = 4
TXT = 256

NC = 2
NS = 16
NW = NC * NS

NPW = (N + NW - 1) // NW
EB = 128
EROWS = 2560
EPAD = EROWS * EB
RPT = EROWS // NS
SLABS = RPT // 16

DHALF = 5000
EACC = 5248
EDUMMY = 5120
ETR = EACC // NS

CR = 1024
CACC = 4224
CDUMMY = 4096
CTR = CACC // NS
NPD = 10 * CR

_MESH = plsc.VectorSubcoreMesh(core_axis_name="c", subcore_axis_name="s")


def _nrsqrt(t):
    y = lax.bitcast_convert_type(t, jnp.int32)
    y = jnp.int32(0x5F3759DF) - (y >> 1)
    y = lax.bitcast_convert_type(y, jnp.float32)
    for _ in range(3):
        y = y * (1.5 - 0.5 * t * y * y)
    return y


@functools.partial(
    pl.kernel,
    mesh=_MESH,
    out_type=jax.ShapeDtypeStruct((N, L, 16), jnp.float32),
    scratch_types=[
        pltpu.VMEM((L,), jnp.int32),
        pltpu.VMEM((L,), jnp.int32),
        pltpu.VMEM((L, EMB), jnp.float32),
        pltpu.VMEM((L, EMB), jnp.float32),
        pltpu.VMEM((L, 16), jnp.float32),
        pltpu.VMEM((L, 16), jnp.float32),
        pltpu.VMEM((EMB,), jnp.float32),
        pltpu.SemaphoreType.DMA,
        pltpu.SemaphoreType.DMA,
        pltpu.SemaphoreType.DMA,
        pltpu.SemaphoreType.DMA,
    ],
)
def _sc_embed(tab_hbm, src_hbm, w_hbm, out_hbm,
              idx0, idx1, S0, S1, P0, P1, w_v, semS0, semS1, semP0, semP1):
    core = lax.axis_index("c")
    sub = lax.axis_index("s")
    wid = core * NS + sub
    base = wid * NPW
    num = jnp.maximum(0, jnp.minimum(NPW, N - base))
    idx = [idx0, idx1]
    S = [S0, S1]
    P = [P0, P1]
    semS = [semS0, semS1]
    semP = [semP0, semP1]

    pltpu.sync_copy(w_hbm, w_v)

    def load_idx_and_gather(i, p):
        pltpu.sync_copy(src_hbm.at[pl.ds((base + i) * L, L)], idx[p])
        pltpu.async_copy(tab_hbm.at[idx[p]], S[p], semS[p])

    @pl.when(num > 0)
    def _():
        load_idx_and_gather(0, 0)

    @pl.when(num > 1)
    def _():
        load_idx_and_gather(1, 1)

    def pair_body(k, carry):
        for p in range(2):
            i = k * 2 + p

            @pl.when(i < num)
            def _():
                node = base + i
                Sp, Pp = S[p], P[p]
                pltpu.make_async_copy(tab_hbm.at[idx[p]], Sp, semS[p]).wait()

                @pl.when(i >= 2)
                def _():
                    pltpu.make_async_copy(Pp, out_hbm.at[node], semP[p]).wait()

                def ss_row(r2, accs):
                    out = accs
                    for rr in range(2):
                        r = r2 * 2 + rr
                        out = tuple(
                            out[j] + Sp[r, pl.ds(j * 16, 16)] * Sp[r, pl.ds(j * 16, 16)]
                            for j in range(8)
                        )
                    return out
                accs = lax.fori_loop(0, L // 2, ss_row,
                                     tuple(jnp.zeros((16,), jnp.float32) for _ in range(8)))

                cs = []
                for j in range(8):
                    wv = w_v[pl.ds(j * 16, 16)]
                    t = wv * wv * accs[j]
                    norm = t * _nrsqrt(t)
                    cs.append(wv / jnp.maximum(norm, 1e-12))

                def p_row(r2, carry2):
                    for rr in range(2):
                        r = r2 * 2 + rr
                        pp = Sp[r, pl.ds(0, 16)] * cs[0]
                        for j in range(1, 8):
                            pp = pp + Sp[r, pl.ds(j * 16, 16)] * cs[j]
                        Pp[r] = pp
                    return carry2
                lax.fori_loop(0, L // 2, p_row, 0)

                pltpu.async_copy(Pp, out_hbm.at[node], semP[p])

                @pl.when(i + 2 < num)
                def _():
                    load_idx_and_gather(i + 2, p)
        return carry

    lax.fori_loop(0, (NPW + 1) // 2, pair_body, 0)

    @pl.when(num >= 1)
    def _():
        pltpu.make_async_copy(P[0], out_hbm.at[base], semP[0]).wait()

    @pl.when(num >= 2)
    def _():
        pltpu.make_async_copy(P[1], out_hbm.at[base], semP[1]).wait()


@functools.partial(
    pl.kernel,
    mesh=_MESH,
    out_type=jax.ShapeDtypeStruct((NC, DHALF, EMB), jnp.float32),
    scratch_types=[
        pltpu.VMEM((16, EB), jnp.int32),
        pltpu.VMEM((16, EB), jnp.int32),
        pltpu.VMEM((16, EB), jnp.int32),
        pltpu.VMEM((16, EB), jnp.int32),
        pltpu.VMEM((16, EB), jnp.int32),
        pltpu.VMEM((16, EB), jnp.int32),
        pltpu.VMEM((EB, EMB), jnp.float32),
        pltpu.VMEM((EB, EMB), jnp.float32),
        pltpu.VMEM((EB, EMB), jnp.float32),
        pltpu.VMEM((EB, EMB), jnp.float32),
        pltpu.VMEM_SHARED((EACC, EMB), jnp.float32),
        pltpu.SemaphoreType.DMA,
        pltpu.SemaphoreType.DMA,
        pltpu.SemaphoreType.DMA,
        pltpu.SemaphoreType.DMA,
    ],
)
def _sc_edge(y_hbm, ic_hbm, vg_hbm, sg_hbm, se_hbm, zz_hbm, agg_hbm,
             vg0, vg1, sg0, sg1, se0, se1, rows0, rows1, sc0, sc1,
             acc_sh, semv0, semv1, sems0, sems1):
    core = lax.axis_index("c")
    sub = lax.axis_index("s")
    vg = [vg0, vg1]
    sg = [sg0, sg1]
    se = [se0, se1]
    rows = [rows0, rows1]
    scl = [sc0, sc1]
    semv = [semv0, semv1]
    sems = [sems0, sems1]

    pltpu.sync_copy(zz_hbm.at[pl.ds(sub * ETR, ETR)],
                    acc_sh.at[pl.ds(sub * ETR, ETR)])
    plsc.subcore_barrier()

    def load_slab(s, par):
        pltpu.sync_copy(vg_hbm.at[pl.ds(sub * RPT + s * 16, 16)], vg[par])
        pltpu.sync_copy(sg_hbm.at[pl.ds(sub * RPT + s * 16, 16)], sg[par])
        pltpu.sync_copy(se_hbm.at[core, pl.ds(sub * RPT + s * 16, 16)], se[par])

    def issue(vgref, sgref, p):
        pltpu.async_copy(y_hbm.at[vgref], rows[p], semv[p])
        pltpu.async_copy(ic_hbm.at[sgref], scl[p], sems[p])

    load_slab(0, 0)
    issue(vg0.at[0], sg0.at[0], 0)
    issue(vg0.at[1], sg0.at[1], 1)

    def slab_pair(sp, carry):
        for sq in range(2):
            s = sp * 2 + sq
            vgc, vgn = vg[sq], vg[1 - sq]
            sgc, sgn = sg[sq], sg[1 - sq]
            sec = se[sq]

            @pl.when(s < SLABS - 1)
            def _():
                load_slab(s + 1, 1 - sq)

            def bpair(bp, c2):
                for bq in range(2):
                    b = bp * 2 + bq
                    gb = s * 16 + b
                    pltpu.make_async_copy(y_hbm.at[vgc.at[b]], rows[bq], semv[bq]).wait()
                    pltpu.make_async_copy(ic_hbm.at[sgc.at[b]], scl[bq], sems[bq]).wait()

                    def mrow(r2, c3):
                        for rr in range(2):
                            r = r2 * 2 + rr
                            for j in range(EMB // 16):
                                rows[bq][r, pl.ds(j * 16, 16)] = (
                                    rows[bq][r, pl.ds(j * 16, 16)]
                                    * scl[bq][r, pl.ds(j * 16, 16)])
                        return c3
                    lax.fori_loop(0, EB // 2, mrow, 0)

                    pltpu.sync_copy(rows[bq], acc_sh.at[sec.at[b]], add=True)

                    @pl.when(bp < 7)
                    def _():
                        issue(vgc.at[b + 2], sgc.at[b + 2], bq)

                    @pl.when(jnp.logical_and(bp == 7, gb + 2 < SLABS * 16))
                    def _():
                        issue(vgn.at[bq], sgn.at[bq], bq)
                return c2

            lax.fori_loop(0, 8, bpair, 0)
        return carry

    lax.fori_loop(0, SLABS // 2, slab_pair, 0)

    plsc.subcore_barrier()

    @pl.when(sub < NS - 1)
    def _():
        pltpu.sync_copy(acc_sh.at[pl.ds(sub * ETR, ETR)],
                        agg_hbm.at[core, pl.ds(sub * ETR, ETR)])

    @pl.when(sub == NS - 1)
    def _():
        pltpu.sync_copy(acc_sh.at[pl.ds((NS - 1) * ETR, DHALF - (NS - 1) * ETR)],
                        agg_hbm.at[core, pl.ds((NS - 1) * ETR, DHALF - (NS - 1) * ETR)])


_BN = 1000


def _tc_fold_body(p_ref, g_ref, out_ref):
    out_ref[...] = jnp.dot(p_ref[...], g_ref[...],
                           preferred_element_type=jnp.float32)


def _tc_fold(p_flat, g):
    return pl.pallas_call(
        _tc_fold_body,
        grid=(N // _BN,),
        in_specs=[
            pl.BlockSpec((_BN * 16, L), lambda i: (i, 0)),
            pl.BlockSpec((L, 8), lambda i: (0, 0)),
        ],
        out_specs=pl.BlockSpec((_BN * 16, 8), lambda i: (i, 0)),
        out_shape=jax.ShapeDtypeStruct((N * 16, 8), jnp.float32),
    )(p_flat, g)


def _tc_inv_body(cnt_ref, out_ref):
    cnt = cnt_ref[...]
    for r in range(NREL):
        s = cnt[:, r * 32:(r + 1) * 32]
        inv = 1.0 / jnp.maximum(s, 1.0)
        out_ref[r] = jnp.concatenate([inv, inv, inv, inv], axis=1)


def _tc_inv(cnt_oh):
    return pl.pallas_call(
        _tc_inv_body,
        grid=(N // _BN,),
        in_specs=[pl.BlockSpec((_BN, EMB), lambda i: (i, 0))],
        out_specs=pl.BlockSpec((NREL, _BN, EMB), lambda i: (0, i, 0)),
        out_shape=jax.ShapeDtypeStruct((NREL, N, EMB), jnp.float32),
    )(cnt_oh)


def _tc_dense1_body(fse_ref, seg_ref, st_ref, w1_ref, b1_ref, w2_ref, b2_ref,
                    cw_ref, x0_ref, y_ref):
    f_se = fse_ref[...]
    seg = seg_ref[...]
    st = st_ref[...]
    st2 = st * st
    c0 = jnp.sum((seg == 0).astype(jnp.float32), axis=1, keepdims=True)
    c1 = jnp.sum((seg == 1).astype(jnp.float32), axis=1, keepdims=True)
    c2 = jnp.sum((seg == 2).astype(jnp.float32), axis=1, keepdims=True)
    q = c0 * st2[0:1, :] + c1 * st2[1:2, :] + c2 * st2[2:3, :]
    inv = 1.0 / jnp.maximum(jnp.sqrt(q), 1e-12)
    d0 = jnp.sum(inv * st[0:1, :], axis=1, keepdims=True)
    d1 = jnp.sum(inv * st[1:2, :], axis=1, keepdims=True)
    d2 = jnp.sum(inv * st[2:3, :], axis=1, keepdims=True)
    f_ge = jnp.where(seg == 0, d0, jnp.where(seg == 1, d1, d2))
    f = f_se + f_ge
    h = jnp.maximum(jnp.dot(f, w1_ref[...], preferred_element_type=jnp.float32)
                    + b1_ref[...], 0.0)
    x0 = jnp.dot(h, w2_ref[...], preferred_element_type=jnp.float32) + b2_ref[...]
    x0_ref[...] = x0
    for r in range(NREL):
        y_ref[r] = jnp.dot(x0, cw_ref[r], preferred_element_type=jnp.float32)


def _tc_dense1(f_se, seg, seg_tab_p, q_W1, q_b1, q_W2, q_b2, c1_w):
    return pl.pallas_call(
        _tc_dense1_body,
        grid=(N // _BN,),
        in_specs=[
            pl.BlockSpec((_BN, L), lambda i: (i, 0)),
            pl.BlockSpec((_BN, L), lambda i: (i, 0)),
            pl.BlockSpec((8, EMB), lambda i: (0, 0)),
            pl.BlockSpec((EMB, TXT), lambda i: (0, 0)),
            pl.BlockSpec((1, TXT), lambda i: (0, 0)),
            pl.BlockSpec((TXT, EMB), lambda i: (0, 0)),
            pl.BlockSpec((1, EMB), lambda i: (0, 0)),
            pl.BlockSpec((NREL, EMB, EMB), lambda i: (0, 0, 0)),
        ],
        out_specs=[
            pl.BlockSpec((_BN, EMB), lambda i: (i, 0)),
            pl.BlockSpec((NREL, _BN, EMB), lambda i: (0, i, 0)),
        ],
        out_shape=[
            jax.ShapeDtypeStruct((N, EMB), jnp.float32),
            jax.ShapeDtypeStruct((NREL, N, EMB), jnp.float32),
        ],
    )(f_se, seg, seg_tab_p, q_W1, q_b1, q_W2, q_b2, c1_w)


def _tc_dense2_body(x_ref, agg_ref, root_ref, b_ref, cw_ref, x1_ref, y_ref):
    x1 = jnp.maximum(
        jnp.dot(x_ref[...], root_ref[...], preferred_element_type=jnp.float32)
        + b_ref[...] + agg_ref[...], 0.0)
    x1_ref[...] = x1
    for r in range(NREL):
        y_ref[r] = jnp.dot(x1, cw_ref[r], preferred_element_type=jnp.float32)


def _tc_dense2(x, agg, root, b, c2_w):
    return pl.pallas_call(
        _tc_dense2_body,
        grid=(N // _BN,),
        in_specs=[
            pl.BlockSpec((_BN, EMB), lambda i: (i, 0)),
            pl.BlockSpec((_BN, EMB), lambda i: (i, 0)),
            pl.BlockSpec((EMB, EMB), lambda i: (0, 0)),
            pl.BlockSpec((1, EMB), lambda i: (0, 0)),
            pl.BlockSpec((NREL, EMB, EMB), lambda i: (0, 0, 0)),
        ],
        out_specs=[
            pl.BlockSpec((_BN, EMB), lambda i: (i, 0)),
            pl.BlockSpec((NREL, _BN, EMB), lambda i: (0, i, 0)),
        ],
        out_shape=[
            jax.ShapeDtypeStruct((N, EMB), jnp.float32),
            jax.ShapeDtypeStruct((NREL, N, EMB), jnp.float32),
        ],
    )(x, agg, root, b, c2_w)


def _tc_dense3_body(x_ref, agg_ref, root_ref, b_ref, lw_ref, lb_ref, out_ref):
    x2 = jnp.maximum(
        jnp.dot(x_ref[...], root_ref[...], preferred_element_type=jnp.float32)
        + b_ref[...] + agg_ref[...], 0.0)
    out_ref[...] = (jnp.dot(x2, lw_ref[...], preferred_element_type=jnp.float32)
                    + lb_ref[...])


def _tc_dense3(x, agg, root, b, lin_W, lin_b):
    return pl.pallas_call(
        _tc_dense3_body,
        grid=(N // _BN,),
        in_specs=[
            pl.BlockSpec((_BN, EMB), lambda i: (i, 0)),
            pl.BlockSpec((_BN, EMB), lambda i: (i, 0)),
            pl.BlockSpec((EMB, EMB), lambda i: (0, 0)),
            pl.BlockSpec((1, EMB), lambda i: (0, 0)),
            pl.BlockSpec((EMB, EMB), lambda i: (0, 0)),
            pl.BlockSpec((1, EMB), lambda i: (0, 0)),
        ],
        out_specs=pl.BlockSpec((_BN, EMB), lambda i: (i, 0)),
        out_shape=jax.ShapeDtypeStruct((N, EMB), jnp.float32),
    )(x, agg, root, b, lin_W, lin_b)


def kernel(src_tab, seg_tab, w, q_W1, q_b1, q_W2, q_b2,
           c1_w, c1_root, c1_b, c2_w, c2_root, c2_b, lin_W, lin_b,
           src, seg, edge_index, edge_type):
    src = src.astype(jnp.int32)
    seg = seg.astype(jnp.int32)
    esrc = edge_index[0].astype(jnp.int32)
    edst = edge_index[1].astype(jnp.int32)
    etype = edge_type.astype(jnp.int32)

    padi = jnp.zeros((EPAD - E,), jnp.int32)
    vgidx = jnp.concatenate([etype * N + esrc, padi]).reshape(EROWS, EB)
    sgidx = jnp.concatenate([etype * N + edst, padi]).reshape(EROWS, EB)
    spread = (jnp.arange(EPAD, dtype=jnp.int32) % 5000) * 8
    cgv = (jnp.concatenate([etype, jnp.full((EPAD - E,), 4, jnp.int32)])
           + spread).reshape(EROWS, EB)
    cgs = (spread + 4).reshape(EROWS, EB)
    se_l = [jnp.concatenate(
        [jnp.where((edst >= h * DHALF) & (edst < (h + 1) * DHALF),
                   edst - h * DHALF, EDUMMY),
         jnp.full((EPAD - E,), EDUMMY, jnp.int32)]) for h in range(NC)]
    seidx = jnp.stack(se_l).reshape(NC, EROWS, EB)

    oh8 = jnp.where(jnp.arange(8, dtype=jnp.int32)[:, None] < 4,
                    (jnp.arange(EMB, dtype=jnp.int32)[None, :] // 32
                     == jnp.arange(8, dtype=jnp.int32)[:, None]).astype(jnp.float32),
                    1.0)
    oh_full = jnp.tile(oh8, (5000, 1))
    zz = jnp.zeros((EACC, EMB), jnp.float32)

    seg_tab_p = jnp.pad(seg_tab, ((0, 5), (0, 0)))
    b1 = q_b1.reshape(1, TXT)
    b2 = q_b2.reshape(1, EMB)
    c1_b_r = c1_b.reshape(1, EMB)
    c2_b_r = c2_b.reshape(1, EMB)
    lin_b_r = lin_b.reshape(1, EMB)

    p_out = _sc_embed(src_tab, src.reshape(N * L), w)
    g = (jnp.arange(L, dtype=jnp.int32)[:, None] // 16
         == jnp.arange(8, dtype=jnp.int32)[None, :]).astype(jnp.float32)
    f_se = _tc_fold(p_out.reshape(N * 16, L), g).reshape(N, L)

    cnt_oh = _sc_edge(oh_full, oh_full, cgv, cgs, seidx, zz)
    ic = _tc_inv(cnt_oh.reshape(N, EMB)).reshape(NREL * N, EMB)

    x0, y0 = _tc_dense1(f_se, seg, seg_tab_p, q_W1, b1, q_W2, b2, c1_w)
    agg1 = _sc_edge(y0.reshape(NREL * N, EMB), ic, vgidx, sgidx, seidx, zz)
    x1, y1 = _tc_dense2(x0, agg1.reshape(N, EMB), c1_root, c1_b_r, c2_w)
    agg2 = _sc_edge(y1.reshape(NREL * N, EMB), ic, vgidx, sgidx, seidx, zz)
    return _tc_dense3(x1, agg2.reshape(N, EMB), c2_root, c2_b_r, lin_W, lin_b_r)
